# Initial kernel scaffold; baseline (speedup 1.0000x reference)
import functools

import jax
import jax.numpy as jnp
from jax import lax
from jax.experimental import pallas as pl
from jax.experimental.pallas import tpu as pltpu
from jax.experimental.pallas import tpu_sc as plsc

N = 10000
D = 128
E = 320000

NC = 2
NS = 16
NW = NC * NS
EPW = E // NW
CHUNK = 80
NCHUNK = EPW // CHUNK
ROWS_PT = N // NS
ZCH = 125
NZ = ROWS_PT // ZCH

_mesh = plsc.VectorSubcoreMesh(core_axis_name="c", subcore_axis_name="s")


@functools.partial(
    pl.kernel,
    mesh=_mesh,
    out_type=jax.ShapeDtypeStruct((NC, N, 16), jnp.float32),
    scratch_types=[
        pltpu.VMEM((NCHUNK, CHUNK), jnp.int32),
        pltpu.VMEM((CHUNK, 16), jnp.float32),
        pltpu.VMEM((ZCH, 16), jnp.float32),
        pltpu.VMEM_SHARED((N, 16), jnp.float32),
    ],
)
def _deg_kernel(dst_hbm, out_hbm, idx_v, ones_v, zero_v, acc):
    c = lax.axis_index("c")
    s = lax.axis_index("s")
    wid = c * NS + s

    @pl.loop(0, CHUNK)
    def _(i):
        ones_v[i] = jnp.ones((16,), jnp.float32)

    @pl.loop(0, ZCH)
    def _(i):
        zero_v[i] = jnp.zeros((16,), jnp.float32)

    base = s * ROWS_PT
    for k in range(NZ):
        pltpu.sync_copy(zero_v, acc.at[pl.ds(base + k * ZCH, ZCH)])
    plsc.subcore_barrier()

    pltpu.sync_copy(dst_hbm.at[wid], idx_v)

    @pl.loop(0, NCHUNK)
    def _(ci):
        pltpu.sync_copy(ones_v, acc.at[idx_v.at[ci]], add=True)

    plsc.subcore_barrier()
    for k in range(NZ):
        sl = pl.ds(base + k * ZCH, ZCH)
        pltpu.sync_copy(acc.at[sl], out_hbm.at[c, sl])


@functools.partial(
    pl.kernel,
    mesh=_mesh,
    out_type=jax.ShapeDtypeStruct((NC, N, D), jnp.float32),
    scratch_types=[
        pltpu.VMEM((NCHUNK, CHUNK), jnp.int32),
        pltpu.VMEM((NCHUNK, CHUNK), jnp.int32),
        pltpu.VMEM((CHUNK, D), jnp.float32),
        pltpu.VMEM((ZCH, D), jnp.float32),
        pltpu.VMEM_SHARED((N, D), jnp.float32),
    ],
)
def _scatter_kernel(hs_hbm, src_hbm, dst_hbm, out_hbm,
                    src_v, dst_v, rows_v, zero_v, acc):
    c = lax.axis_index("c")
    s = lax.axis_index("s")
    wid = c * NS + s

    @pl.loop(0, ZCH)
    def _(i):
        for j in range(D // 16):
            zero_v[i, pl.ds(j * 16, 16)] = jnp.zeros((16,), jnp.float32)

    base = s * ROWS_PT
    for k in range(NZ):
        pltpu.sync_copy(zero_v, acc.at[pl.ds(base + k * ZCH, ZCH)])
    plsc.subcore_barrier()

    pltpu.sync_copy(src_hbm.at[wid], src_v)
    pltpu.sync_copy(dst_hbm.at[wid], dst_v)

    @pl.loop(0, NCHUNK)
    def _(ci):
        pltpu.sync_copy(hs_hbm.at[src_v.at[ci]], rows_v)
        pltpu.sync_copy(rows_v, acc.at[dst_v.at[ci]], add=True)

    plsc.subcore_barrier()
    for k in range(NZ):
        sl = pl.ds(base + k * ZCH, ZCH)
        pltpu.sync_copy(acc.at[sl], out_hbm.at[c, sl])


_BLK = 1000
_GRID = (N // _BLK,)


def _norm_from_deg(deg_ref):
    d = 1.0 + deg_ref[0, :, 0] + deg_ref[1, :, 0]
    return lax.rsqrt(d)[:, None]


def _mm1_body(deg_ref, x_ref, w_ref, hs_ref):
    nrm = _norm_from_deg(deg_ref)
    hw = jnp.dot(x_ref[...], w_ref[...], preferred_element_type=jnp.float32)
    hs_ref[...] = hw * nrm


def _mid_body(deg_ref, p_ref, hs_ref, b_ref, w_ref, o_ref):
    nrm = _norm_from_deg(deg_ref)
    agg = p_ref[0] + p_ref[1] + hs_ref[...]
    h = jnp.maximum(agg * nrm + b_ref[...], 0.0)
    o_ref[...] = jnp.dot(h, w_ref[...], preferred_element_type=jnp.float32) * nrm


def _fin_body(deg_ref, p_ref, hs_ref, b_ref, o_ref):
    nrm = _norm_from_deg(deg_ref)
    agg = p_ref[0] + p_ref[1] + hs_ref[...]
    o_ref[...] = agg * nrm + b_ref[...]


_deg_spec = pl.BlockSpec((NC, _BLK, 16), lambda i: (0, i, 0))
_row_spec = pl.BlockSpec((_BLK, D), lambda i: (i, 0))
_p_spec = pl.BlockSpec((NC, _BLK, D), lambda i: (0, i, 0))
_w_spec = pl.BlockSpec((D, D), lambda i: (0, 0))
_b_spec = pl.BlockSpec((1, D), lambda i: (0, 0))
_out_t = jax.ShapeDtypeStruct((N, D), jnp.float32)

_mm1 = pl.pallas_call(
    _mm1_body, grid=_GRID,
    in_specs=[_deg_spec, _row_spec, _w_spec],
    out_specs=_row_spec, out_shape=_out_t)

_mid = pl.pallas_call(
    _mid_body, grid=_GRID,
    in_specs=[_deg_spec, _p_spec, _row_spec, _b_spec, _w_spec],
    out_specs=_row_spec, out_shape=_out_t)

_fin = pl.pallas_call(
    _fin_body, grid=_GRID,
    in_specs=[_deg_spec, _p_spec, _row_spec, _b_spec],
    out_specs=_row_spec, out_shape=_out_t)


def kernel(x, edge_index, W1, b1, W2, b2, W3, b3):
    src3 = edge_index[0].reshape(NW, NCHUNK, CHUNK)
    dst3 = edge_index[1].reshape(NW, NCHUNK, CHUNK)
    b1r = b1.reshape(1, D)
    b2r = b2.reshape(1, D)
    b3r = b3.reshape(1, D)

    degp = _deg_kernel(dst3)
    hs1 = _mm1(degp, x, W1)
    p1 = _scatter_kernel(hs1, src3, dst3)
    hs2 = _mid(degp, p1, hs1, b1r, W2)
    p2 = _scatter_kernel(hs2, src3, dst3)
    hs3 = _mid(degp, p2, hs2, b2r, W3)
    p3 = _scatter_kernel(hs3, src3, dst3)
    return _fin(degp, p3, hs3, b3r)


# speedup vs baseline: 16.0005x; 16.0005x over previous
import functools

import jax
import jax.numpy as jnp
from jax import lax
from jax.experimental import pallas as pl
from jax.experimental.pallas import tpu as pltpu
from jax.experimental.pallas import tpu_sc as plsc

N = 10000
D = 128
E = 320000

NC = 2
NS = 16
NW = NC * NS
EPW = E // NW
CHUNK = 80
NCHUNK = EPW // CHUNK
NP = 10240
ROWS_PT = NP // NS
ZCH = CHUNK
NZ = ROWS_PT // ZCH
CCH = 128
NCP = ROWS_PT // CCH

_mesh = plsc.VectorSubcoreMesh(core_axis_name="c", subcore_axis_name="s")


@functools.partial(
    pl.kernel,
    mesh=_mesh,
    out_type=jax.ShapeDtypeStruct((NC, NP, D), jnp.float32),
    scratch_types=[
        pltpu.VMEM((NCHUNK, CHUNK), jnp.int32),
        pltpu.VMEM((CHUNK, D), jnp.float32),
        pltpu.VMEM_SHARED((NP, D), jnp.float32),
    ],
)
def _deg_kernel(dst_hbm, out_hbm, idx_v, ones_v, acc):
    c = lax.axis_index("c")
    s = lax.axis_index("s")
    wid = c * NS + s

    @pl.loop(0, CHUNK)
    def _(i):
        for j in range(D // 16):
            ones_v[i, pl.ds(j * 16, 16)] = jnp.zeros((16,), jnp.float32)

    base = s * ROWS_PT
    for k in range(NZ):
        pltpu.sync_copy(ones_v, acc.at[pl.ds(base + k * ZCH, ZCH)])

    @pl.loop(0, CHUNK)
    def _(i):
        for j in range(D // 16):
            ones_v[i, pl.ds(j * 16, 16)] = jnp.ones((16,), jnp.float32)
    plsc.subcore_barrier()

    pltpu.sync_copy(dst_hbm.at[wid], idx_v)

    @pl.loop(0, NCHUNK)
    def _(ci):
        pltpu.sync_copy(ones_v, acc.at[idx_v.at[ci]], add=True)

    plsc.subcore_barrier()
    for k in range(NCP):
        sl = pl.ds(base + k * CCH, CCH)
        pltpu.sync_copy(acc.at[sl], out_hbm.at[c, sl])


@functools.partial(
    pl.kernel,
    mesh=_mesh,
    out_type=jax.ShapeDtypeStruct((NC, NP, D), jnp.float32),
    scratch_types=[
        pltpu.VMEM((NCHUNK, CHUNK), jnp.int32),
        pltpu.VMEM((NCHUNK, CHUNK), jnp.int32),
        pltpu.VMEM((CHUNK, D), jnp.float32),
        pltpu.VMEM_SHARED((NP, D), jnp.float32),
    ],
)
def _scatter_kernel(hs_hbm, src_hbm, dst_hbm, out_hbm,
                    src_v, dst_v, rows_v, acc):
    c = lax.axis_index("c")
    s = lax.axis_index("s")
    wid = c * NS + s

    @pl.loop(0, CHUNK)
    def _(i):
        for j in range(D // 16):
            rows_v[i, pl.ds(j * 16, 16)] = jnp.zeros((16,), jnp.float32)

    base = s * ROWS_PT
    for k in range(NZ):
        pltpu.sync_copy(rows_v, acc.at[pl.ds(base + k * ZCH, ZCH)])
    plsc.subcore_barrier()

    pltpu.sync_copy(src_hbm.at[wid], src_v)
    pltpu.sync_copy(dst_hbm.at[wid], dst_v)

    @pl.loop(0, NCHUNK)
    def _(ci):
        pltpu.sync_copy(hs_hbm.at[src_v.at[ci]], rows_v)
        pltpu.sync_copy(rows_v, acc.at[dst_v.at[ci]], add=True)

    plsc.subcore_barrier()
    for k in range(NCP):
        sl = pl.ds(base + k * CCH, CCH)
        pltpu.sync_copy(acc.at[sl], out_hbm.at[c, sl])


_BLK = 1000
_GRID = (N // _BLK,)


def _norm_from_deg(deg_ref):
    d = 1.0 + deg_ref[0, :, 0] + deg_ref[1, :, 0]
    return lax.rsqrt(d)[:, None]


def _mm1_body(deg_ref, x_ref, w_ref, hs_ref):
    nrm = _norm_from_deg(deg_ref)
    hw = jnp.dot(x_ref[...], w_ref[...], preferred_element_type=jnp.float32)
    hs_ref[...] = hw * nrm


def _mid_body(deg_ref, p_ref, hs_ref, b_ref, w_ref, o_ref):
    nrm = _norm_from_deg(deg_ref)
    agg = p_ref[0] + p_ref[1] + hs_ref[...]
    h = jnp.maximum(agg * nrm + b_ref[...], 0.0)
    o_ref[...] = jnp.dot(h, w_ref[...], preferred_element_type=jnp.float32) * nrm


def _fin_body(deg_ref, p_ref, hs_ref, b_ref, o_ref):
    nrm = _norm_from_deg(deg_ref)
    agg = p_ref[0] + p_ref[1] + hs_ref[...]
    o_ref[...] = agg * nrm + b_ref[...]


_deg_spec = pl.BlockSpec((NC, _BLK, D), lambda i: (0, i, 0))
_row_spec = pl.BlockSpec((_BLK, D), lambda i: (i, 0))
_p_spec = pl.BlockSpec((NC, _BLK, D), lambda i: (0, i, 0))
_w_spec = pl.BlockSpec((D, D), lambda i: (0, 0))
_b_spec = pl.BlockSpec((1, D), lambda i: (0, 0))
_out_t = jax.ShapeDtypeStruct((N, D), jnp.float32)

_mm1 = pl.pallas_call(
    _mm1_body, grid=_GRID,
    in_specs=[_deg_spec, _row_spec, _w_spec],
    out_specs=_row_spec, out_shape=_out_t)

_mid = pl.pallas_call(
    _mid_body, grid=_GRID,
    in_specs=[_deg_spec, _p_spec, _row_spec, _b_spec, _w_spec],
    out_specs=_row_spec, out_shape=_out_t)

_fin = pl.pallas_call(
    _fin_body, grid=_GRID,
    in_specs=[_deg_spec, _p_spec, _row_spec, _b_spec],
    out_specs=_row_spec, out_shape=_out_t)


def kernel(x, edge_index, W1, b1, W2, b2, W3, b3):
    src3 = edge_index[0].reshape(NW, NCHUNK, CHUNK)
    dst3 = edge_index[1].reshape(NW, NCHUNK, CHUNK)
    b1r = b1.reshape(1, D)
    b2r = b2.reshape(1, D)
    b3r = b3.reshape(1, D)

    degp = _deg_kernel(dst3)
    hs1 = _mm1(degp, x, W1)
    p1 = _scatter_kernel(hs1, src3, dst3)
    hs2 = _mid(degp, p1, hs1, b1r, W2)
    p2 = _scatter_kernel(hs2, src3, dst3)
    hs3 = _mid(degp, p2, hs2, b2r, W3)
    p3 = _scatter_kernel(hs3, src3, dst3)
    return _fin(degp, p3, hs3, b3r)


# speedup vs baseline: 22.0375x; 1.3773x over previous
import functools

import jax
import jax.numpy as jnp
from jax import lax
from jax.experimental import pallas as pl
from jax.experimental.pallas import tpu as pltpu
from jax.experimental.pallas import tpu_sc as plsc

N = 10000
D = 128
E = 320000

NC = 2
NS = 16
NW = NC * NS
EPW = E // NW
CHUNK = 125
NCHUNK = EPW // CHUNK
SB = 16
NSB = NCHUNK // SB
NP = 10112
ROWS_PT = NP // NS
ZCHUNKS = ((0, 120), (120, 120), (240, 120), (360, 120), (480, 120), (600, 32))

_mesh = plsc.VectorSubcoreMesh(core_axis_name="c", subcore_axis_name="s")


@functools.partial(
    pl.kernel,
    mesh=_mesh,
    out_type=jax.ShapeDtypeStruct((NC, NP, D), jnp.float32),
    scratch_types=[
        pltpu.VMEM((NCHUNK, CHUNK), jnp.int32),
        pltpu.VMEM((CHUNK, D), jnp.float32),
        pltpu.VMEM_SHARED((NP, D), jnp.float32),
    ],
)
def _deg_kernel(dst_hbm, out_hbm, idx_v, ones_v, acc):
    c = lax.axis_index("c")
    s = lax.axis_index("s")
    wid = c * NS + s

    @pl.loop(0, CHUNK)
    def _(i):
        for j in range(D // 16):
            ones_v[i, pl.ds(j * 16, 16)] = jnp.zeros((16,), jnp.float32)

    base = s * ROWS_PT
    for off, ln in ZCHUNKS:
        pltpu.sync_copy(ones_v.at[pl.ds(0, ln)], acc.at[pl.ds(base + off, ln)])

    @pl.loop(0, CHUNK)
    def _(i):
        for j in range(D // 16):
            ones_v[i, pl.ds(j * 16, 16)] = jnp.ones((16,), jnp.float32)
    plsc.subcore_barrier()

    pltpu.sync_copy(dst_hbm.at[wid], idx_v)

    @pl.loop(0, NCHUNK)
    def _(ci):
        pltpu.sync_copy(ones_v, acc.at[idx_v.at[ci]], add=True)

    plsc.subcore_barrier()
    for off, ln in ZCHUNKS:
        sl = pl.ds(base + off, ln)
        pltpu.sync_copy(acc.at[sl], out_hbm.at[c, sl])


@functools.partial(
    pl.kernel,
    mesh=_mesh,
    out_type=jax.ShapeDtypeStruct((NC, NP, D), jnp.float32),
    scratch_types=[
        pltpu.VMEM((SB, CHUNK), jnp.int32),
        pltpu.VMEM((SB, CHUNK), jnp.int32),
        pltpu.VMEM((2, CHUNK, D), jnp.float32),
        pltpu.VMEM_SHARED((NP, D), jnp.float32),
        pltpu.SemaphoreType.DMA,
    ],
)
def _scatter_kernel(hs_hbm, src_hbm, dst_hbm, out_hbm,
                    src_v, dst_v, rows_v, acc, gsem):
    c = lax.axis_index("c")
    s = lax.axis_index("s")
    wid = c * NS + s

    @pl.loop(0, CHUNK)
    def _(i):
        for j in range(D // 16):
            rows_v[0, i, pl.ds(j * 16, 16)] = jnp.zeros((16,), jnp.float32)

    base = s * ROWS_PT
    for off, ln in ZCHUNKS:
        pltpu.sync_copy(rows_v.at[0, pl.ds(0, ln)], acc.at[pl.ds(base + off, ln)])
    plsc.subcore_barrier()

    @pl.loop(0, NSB)
    def _(sb):
        sl = pl.ds(sb * SB, SB)
        pltpu.sync_copy(src_hbm.at[wid, sl], src_v)
        pltpu.sync_copy(dst_hbm.at[wid, sl], dst_v)
        pltpu.async_copy(hs_hbm.at[src_v.at[0]], rows_v.at[0], gsem)
        for k in range(SB):
            b = k % 2
            pltpu.make_async_copy(
                hs_hbm.at[src_v.at[k]], rows_v.at[b], gsem).wait()
            if k + 1 < SB:
                pltpu.async_copy(
                    hs_hbm.at[src_v.at[k + 1]], rows_v.at[1 - b], gsem)
            pltpu.sync_copy(rows_v.at[b], acc.at[dst_v.at[k]], add=True)

    plsc.subcore_barrier()
    for off, ln in ZCHUNKS:
        sl = pl.ds(base + off, ln)
        pltpu.sync_copy(acc.at[sl], out_hbm.at[c, sl])


_BLK = 1000
_GRID = (N // _BLK,)


def _norm_from_deg(deg_ref):
    d = 1.0 + deg_ref[0, :, 0] + deg_ref[1, :, 0]
    return lax.rsqrt(d)[:, None]


def _mm1_body(deg_ref, x_ref, w_ref, hs_ref):
    nrm = _norm_from_deg(deg_ref)
    hw = jnp.dot(x_ref[...], w_ref[...], preferred_element_type=jnp.float32)
    hs_ref[...] = hw * nrm


def _mid_body(deg_ref, p_ref, hs_ref, b_ref, w_ref, o_ref):
    nrm = _norm_from_deg(deg_ref)
    agg = p_ref[0] + p_ref[1] + hs_ref[...]
    h = jnp.maximum(agg * nrm + b_ref[...], 0.0)
    o_ref[...] = jnp.dot(h, w_ref[...], preferred_element_type=jnp.float32) * nrm


def _fin_body(deg_ref, p_ref, hs_ref, b_ref, o_ref):
    nrm = _norm_from_deg(deg_ref)
    agg = p_ref[0] + p_ref[1] + hs_ref[...]
    o_ref[...] = agg * nrm + b_ref[...]


_deg_spec = pl.BlockSpec((NC, _BLK, D), lambda i: (0, i, 0))
_row_spec = pl.BlockSpec((_BLK, D), lambda i: (i, 0))
_p_spec = pl.BlockSpec((NC, _BLK, D), lambda i: (0, i, 0))
_w_spec = pl.BlockSpec((D, D), lambda i: (0, 0))
_b_spec = pl.BlockSpec((1, D), lambda i: (0, 0))
_out_t = jax.ShapeDtypeStruct((N, D), jnp.float32)

_mm1 = pl.pallas_call(
    _mm1_body, grid=_GRID,
    in_specs=[_deg_spec, _row_spec, _w_spec],
    out_specs=_row_spec, out_shape=_out_t)

_mid = pl.pallas_call(
    _mid_body, grid=_GRID,
    in_specs=[_deg_spec, _p_spec, _row_spec, _b_spec, _w_spec],
    out_specs=_row_spec, out_shape=_out_t)

_fin = pl.pallas_call(
    _fin_body, grid=_GRID,
    in_specs=[_deg_spec, _p_spec, _row_spec, _b_spec],
    out_specs=_row_spec, out_shape=_out_t)


def kernel(x, edge_index, W1, b1, W2, b2, W3, b3):
    src3 = edge_index[0].reshape(NW, NCHUNK, CHUNK)
    dst3 = edge_index[1].reshape(NW, NCHUNK, CHUNK)
    b1r = b1.reshape(1, D)
    b2r = b2.reshape(1, D)
    b3r = b3.reshape(1, D)

    degp = _deg_kernel(dst3)
    hs1 = _mm1(degp, x, W1)
    p1 = _scatter_kernel(hs1, src3, dst3)
    hs2 = _mid(degp, p1, hs1, b1r, W2)
    p2 = _scatter_kernel(hs2, src3, dst3)
    hs3 = _mid(degp, p2, hs2, b2r, W3)
    p3 = _scatter_kernel(hs3, src3, dst3)
    return _fin(degp, p3, hs3, b3r)


# speedup vs baseline: 22.1364x; 1.0045x over previous
import functools

import jax
import jax.numpy as jnp
from jax import lax
from jax.experimental import pallas as pl
from jax.experimental.pallas import tpu as pltpu
from jax.experimental.pallas import tpu_sc as plsc

N = 10000
D = 128
E = 320000

NC = 2
NS = 16
NW = NC * NS
EPW = E // NW
CHUNK = 125
NCHUNK = EPW // CHUNK
SB = 16
NSB = NCHUNK // SB
NP = 10112
ROWS_PT = NP // NS
ZCHUNKS = ((0, 120), (120, 120), (240, 120), (360, 120), (480, 120), (600, 32))

_mesh = plsc.VectorSubcoreMesh(core_axis_name="c", subcore_axis_name="s")


@functools.partial(
    pl.kernel,
    mesh=_mesh,
    out_type=jax.ShapeDtypeStruct((NC, NP, D), jnp.float32),
    scratch_types=[
        pltpu.VMEM((NCHUNK, CHUNK), jnp.int32),
        pltpu.VMEM((CHUNK, D), jnp.float32),
        pltpu.VMEM_SHARED((NP, D), jnp.float32),
        pltpu.SemaphoreType.DMA,
    ],
)
def _deg_kernel(dst_hbm, out_hbm, idx_v, ones_v, acc, zsem):
    c = lax.axis_index("c")
    s = lax.axis_index("s")
    wid = c * NS + s

    @pl.loop(0, CHUNK)
    def _(i):
        for j in range(D // 16):
            ones_v[i, pl.ds(j * 16, 16)] = jnp.zeros((16,), jnp.float32)

    base = s * ROWS_PT
    for off, ln in ZCHUNKS:
        pltpu.async_copy(ones_v.at[pl.ds(0, ln)], acc.at[pl.ds(base + off, ln)], zsem)
    for off, ln in ZCHUNKS:
        pltpu.make_async_copy(
            ones_v.at[pl.ds(0, ln)], acc.at[pl.ds(base + off, ln)], zsem).wait()

    @pl.loop(0, CHUNK)
    def _(i):
        for j in range(D // 16):
            ones_v[i, pl.ds(j * 16, 16)] = jnp.ones((16,), jnp.float32)
    plsc.subcore_barrier()

    pltpu.sync_copy(dst_hbm.at[wid], idx_v)

    @pl.loop(0, NCHUNK)
    def _(ci):
        pltpu.sync_copy(ones_v, acc.at[idx_v.at[ci]], add=True)

    plsc.subcore_barrier()
    sl = pl.ds(base, ROWS_PT)
    pltpu.sync_copy(acc.at[sl], out_hbm.at[c, sl])


@functools.partial(
    pl.kernel,
    mesh=_mesh,
    out_type=jax.ShapeDtypeStruct((NC, NP, D), jnp.float32),
    scratch_types=[
        pltpu.VMEM((SB, CHUNK), jnp.int32),
        pltpu.VMEM((SB, CHUNK), jnp.int32),
        pltpu.VMEM((2, CHUNK, D), jnp.float32),
        pltpu.VMEM_SHARED((NP, D), jnp.float32),
        pltpu.SemaphoreType.DMA,
    ],
)
def _scatter_kernel(hs_hbm, src_hbm, dst_hbm, out_hbm,
                    src_v, dst_v, rows_v, acc, gsem):
    c = lax.axis_index("c")
    s = lax.axis_index("s")
    wid = c * NS + s

    @pl.loop(0, CHUNK)
    def _(i):
        for j in range(D // 16):
            rows_v[0, i, pl.ds(j * 16, 16)] = jnp.zeros((16,), jnp.float32)

    base = s * ROWS_PT
    for off, ln in ZCHUNKS:
        pltpu.async_copy(rows_v.at[0, pl.ds(0, ln)], acc.at[pl.ds(base + off, ln)], gsem)
    for off, ln in ZCHUNKS:
        pltpu.make_async_copy(
            rows_v.at[0, pl.ds(0, ln)], acc.at[pl.ds(base + off, ln)], gsem).wait()
    plsc.subcore_barrier()

    @pl.loop(0, NSB)
    def _(sb):
        sl = pl.ds(sb * SB, SB)
        pltpu.sync_copy(src_hbm.at[wid, sl], src_v)
        pltpu.sync_copy(dst_hbm.at[wid, sl], dst_v)
        pltpu.async_copy(hs_hbm.at[src_v.at[0]], rows_v.at[0], gsem)
        for k in range(SB):
            b = k % 2
            pltpu.make_async_copy(
                hs_hbm.at[src_v.at[k]], rows_v.at[b], gsem).wait()
            if k + 1 < SB:
                pltpu.async_copy(
                    hs_hbm.at[src_v.at[k + 1]], rows_v.at[1 - b], gsem)
            pltpu.sync_copy(rows_v.at[b], acc.at[dst_v.at[k]], add=True)

    plsc.subcore_barrier()
    sl = pl.ds(base, ROWS_PT)
    pltpu.sync_copy(acc.at[sl], out_hbm.at[c, sl])


_BLK = 1000
_GRID = (N // _BLK,)


def _norm_from_deg(deg_ref):
    d = 1.0 + deg_ref[0, :, 0] + deg_ref[1, :, 0]
    return lax.rsqrt(d)[:, None]


def _mm1_body(x_ref, w_ref, hw_ref):
    hw_ref[...] = jnp.dot(x_ref[...], w_ref[...],
                          preferred_element_type=jnp.float32)


def _scale_body(deg_ref, hw_ref, hs_ref):
    hs_ref[...] = hw_ref[...] * _norm_from_deg(deg_ref)


def _mid_body(deg_ref, p_ref, hs_ref, b_ref, w_ref, o_ref):
    nrm = _norm_from_deg(deg_ref)
    agg = p_ref[0] + p_ref[1] + hs_ref[...]
    h = jnp.maximum(agg * nrm + b_ref[...], 0.0)
    o_ref[...] = jnp.dot(h, w_ref[...], preferred_element_type=jnp.float32) * nrm


def _fin_body(deg_ref, p_ref, hs_ref, b_ref, o_ref):
    nrm = _norm_from_deg(deg_ref)
    agg = p_ref[0] + p_ref[1] + hs_ref[...]
    o_ref[...] = agg * nrm + b_ref[...]


_deg_spec = pl.BlockSpec((NC, _BLK, D), lambda i: (0, i, 0))
_row_spec = pl.BlockSpec((_BLK, D), lambda i: (i, 0))
_p_spec = pl.BlockSpec((NC, _BLK, D), lambda i: (0, i, 0))
_w_spec = pl.BlockSpec((D, D), lambda i: (0, 0))
_b_spec = pl.BlockSpec((1, D), lambda i: (0, 0))
_out_t = jax.ShapeDtypeStruct((N, D), jnp.float32)

_mm1 = pl.pallas_call(
    _mm1_body, grid=_GRID,
    in_specs=[_row_spec, _w_spec],
    out_specs=_row_spec, out_shape=_out_t)

_scale = pl.pallas_call(
    _scale_body, grid=_GRID,
    in_specs=[_deg_spec, _row_spec],
    out_specs=_row_spec, out_shape=_out_t)

_mid = pl.pallas_call(
    _mid_body, grid=_GRID,
    in_specs=[_deg_spec, _p_spec, _row_spec, _b_spec, _w_spec],
    out_specs=_row_spec, out_shape=_out_t)

_fin = pl.pallas_call(
    _fin_body, grid=_GRID,
    in_specs=[_deg_spec, _p_spec, _row_spec, _b_spec],
    out_specs=_row_spec, out_shape=_out_t)


def kernel(x, edge_index, W1, b1, W2, b2, W3, b3):
    src3 = edge_index[0].reshape(NW, NCHUNK, CHUNK)
    dst3 = edge_index[1].reshape(NW, NCHUNK, CHUNK)
    b1r = b1.reshape(1, D)
    b2r = b2.reshape(1, D)
    b3r = b3.reshape(1, D)

    degp = _deg_kernel(dst3)
    hw1 = _mm1(x, W1)
    hs1 = _scale(degp, hw1)
    p1 = _scatter_kernel(hs1, src3, dst3)
    hs2 = _mid(degp, p1, hs1, b1r, W2)
    p2 = _scatter_kernel(hs2, src3, dst3)
    hs3 = _mid(degp, p2, hs2, b2r, W3)
    p3 = _scatter_kernel(hs3, src3, dst3)
    return _fin(degp, p3, hs3, b3r)
